# baseline (speedup 1.0000x reference)
import functools
import math

import jax
import jax.numpy as jnp
from jax import lax
from jax.experimental import pallas as pl
from jax.experimental.pallas import tpu as pltpu
from jax.experimental.pallas import tpu_sc as plsc

_D = 1024
_LANES = 16
_MAX_TIMESCALE = 10000.0
_SCALE = math.sqrt(_D)

_NC = 2
_NS = 16
_NW = _NC * _NS

_CHUNK = 16
_NCHUNK = 16
_ROWS_PER_W = _CHUNK * _NCHUNK


def _pe_table(seq):
    inc = math.log(_MAX_TIMESCALE) / _D
    inv_timescales = jnp.exp(
        jnp.arange(0, _D, 2, dtype=jnp.float32) * -inc)
    position = jnp.arange(0, seq, dtype=jnp.float32)[:, None]
    pe = jnp.zeros((seq, _D), dtype=jnp.float32)
    pe = pe.at[:, 0::2].set(jnp.sin(position * inv_timescales))
    pe = pe.at[:, 1::2].set(jnp.cos(position * inv_timescales))
    return pe


def _sc_embed(table, idx3, pe, seq):
    n_rows = _NW * _ROWS_PER_W
    w_per_batch = seq // _ROWS_PER_W
    mesh = plsc.VectorSubcoreMesh(core_axis_name="c", subcore_axis_name="s")

    @functools.partial(
        pl.kernel,
        mesh=mesh,
        out_type=jax.ShapeDtypeStruct((n_rows, _D), jnp.float32),
        scratch_types=[
            pltpu.VMEM((_NCHUNK, _CHUNK), jnp.int32),
            pltpu.VMEM((_CHUNK, _D), jnp.float32),
            pltpu.VMEM((_CHUNK, _D), jnp.float32),
            pltpu.VMEM((_CHUNK, _D), jnp.float32),
            pltpu.VMEM((_CHUNK, _D), jnp.float32),
            pltpu.SemaphoreType.DMA,
            pltpu.SemaphoreType.DMA,
            pltpu.SemaphoreType.DMA,
            pltpu.SemaphoreType.DMA,
            pltpu.SemaphoreType.DMA,
            pltpu.SemaphoreType.DMA,
        ],
    )
    def k(table_hbm, idx_hbm, pe_hbm, out_hbm,
          idx_v, rows0, rows1, pe0, pe1, g0, g1, p0, p1, w0, w1):
        wid = lax.axis_index("s") * _NC + lax.axis_index("c")
        base = wid * _ROWS_PER_W
        pbase = lax.rem(wid, w_per_batch) * _ROWS_PER_W
        rows = (rows0, rows1)
        peb = (pe0, pe1)
        gsem = (g0, g1)
        psem = (p0, p1)
        wsem = (w0, w1)

        pltpu.sync_copy(idx_hbm.at[wid], idx_v)

        def issue(c, b):
            gcp = pltpu.async_copy(table_hbm.at[idx_v.at[c]], rows[b], gsem[b])
            pcp = pltpu.async_copy(
                pe_hbm.at[pl.ds(pbase + c * _CHUNK, _CHUNK)], peb[b], psem[b])
            return gcp, pcp

        gp = [None, None]
        wcp = [None, None]
        gp[0] = issue(0, 0)
        for c in range(_NCHUNK):
            b = c % 2
            nb = 1 - b
            if c + 1 < _NCHUNK:
                if wcp[nb] is not None:
                    wcp[nb].wait()
                gp[nb] = issue(c + 1, nb)
            gp[b][0].wait()
            gp[b][1].wait()

            rb = rows[b]
            pb = peb[b]

            @pl.loop(0, _CHUNK)
            def _(r):
                for col in range(_D // _LANES):
                    slc = pl.ds(col * _LANES, _LANES)
                    rb[r, slc] = rb[r, slc] * _SCALE + pb[r, slc]

            wcp[b] = pltpu.async_copy(
                rb, out_hbm.at[pl.ds(base + c * _CHUNK, _CHUNK)], wsem[b])
        wcp[0].wait()
        wcp[1].wait()

    return k(table, idx3, pe)


def kernel(x, table):
    batch, seq = x.shape
    n_rows = batch * seq
    assert n_rows == _NW * _ROWS_PER_W

    idx3 = x.reshape(_NW, _NCHUNK, _CHUNK)
    pe = _pe_table(seq)
    out = _sc_embed(table, idx3, pe, seq)
    return out.reshape(batch, seq, _D)
